# baseline (speedup 1.0000x reference)
import functools

import jax
import jax.numpy as jnp
from jax import lax
from jax.experimental import pallas as pl
from jax.experimental.pallas import tpu as pltpu
from jax.experimental.pallas import tpu_sc as plsc

N = 10000
NPP = 10240
E = 320000
D = 128
CHUNK = 128
NTILES = 32
RPT = 79
NCHP = NTILES * RPT
E_PAD = NCHP * CHUNK
ROWS_PER_TILE = NPP // 16
DEGR = NPP // 16


def _make_sc_agg(with_deg):
    mesh = plsc.VectorSubcoreMesh(core_axis_name="c", subcore_axis_name="s",
                                  num_cores=2, num_subcores=16)

    out_type = [jax.ShapeDtypeStruct((2 * NPP, D), jnp.float32)]
    scratch = [
        pltpu.VMEM((4, 2, CHUNK), jnp.int32),
        pltpu.VMEM((CHUNK, D), jnp.float32),
        pltpu.VMEM((CHUNK, D), jnp.float32),
        pltpu.VMEM_SHARED((NPP, D), jnp.float32),
        pltpu.SemaphoreType.DMA,
        pltpu.SemaphoreType.DMA,
        pltpu.SemaphoreType.DMA,
        pltpu.SemaphoreType.DMA,
        pltpu.SemaphoreType.DMA,
        pltpu.SemaphoreType.DMA,
    ]
    if with_deg:
        out_type.append(jax.ShapeDtypeStruct((2 * DEGR, 16), jnp.float32))
        scratch.append(pltpu.VMEM((DEGR, 16), jnp.float32))
        scratch.append(pltpu.VMEM((DEGR,), jnp.int32))
        scratch.append(pltpu.VMEM_SHARED((DEGR, 16), jnp.float32))

    @functools.partial(pl.kernel, out_type=out_type, mesh=mesh,
                       scratch_types=scratch,
                       compiler_params=pltpu.CompilerParams(
                           use_tc_tiling_on_sc=False,
                           needs_layout_passes=False))
    def sc_agg(x_hbm, edges_hbm, zeros_hbm, zdeg_hbm, iota_hbm, *rest):
        if with_deg:
            (out_hbm, deg_hbm, ebuf, rows0, rows1, acc_sh,
             gsem0, gsem1, ssem0, ssem1, isem0, isem1,
             degl, iota_v, deg_sh) = rest
        else:
            (out_hbm, ebuf, rows0, rows1, acc_sh,
             gsem0, gsem1, ssem0, ssem1, isem0, isem1) = rest
        c = lax.axis_index("c")
        s = lax.axis_index("s")
        wid = c * 16 + s
        base = wid * RPT

        pltpu.sync_copy(zeros_hbm, acc_sh.at[pl.ds(s * ROWS_PER_TILE, ROWS_PER_TILE)])
        if with_deg:
            pltpu.sync_copy(zdeg_hbm, degl)
            pltpu.sync_copy(iota_hbm, iota_v)
            pltpu.sync_copy(zdeg_hbm.at[pl.ds(s * (DEGR // 16), DEGR // 16)],
                            deg_sh.at[pl.ds(s * (DEGR // 16), DEGR // 16)])
        plsc.subcore_barrier()

        rows = (rows0, rows1)
        gsem = (gsem0, gsem1)
        ssem = (ssem0, ssem1)
        isem = (isem0, isem1)
        ones16 = jnp.full((16,), 1.0, jnp.float32)

        def i_start(j, slot, sem):
            pltpu.async_copy(edges_hbm.at[base + j], ebuf.at[slot], sem)

        def i_wait(sem):
            pltpu.make_async_copy(edges_hbm.at[base], ebuf.at[0], sem).wait()

        def g_start(slot, b):
            pltpu.async_copy(x_hbm.at[ebuf.at[slot, 0]], rows[b], gsem[b])

        def g_wait(b):
            pltpu.make_async_copy(x_hbm.at[ebuf.at[0, 0]], rows[b], gsem[b]).wait()

        def s_start(slot, b):
            pltpu.async_copy(rows[b], acc_sh.at[ebuf.at[slot, 1]], ssem[b], add=True)

        def s_wait(b):
            pltpu.make_async_copy(rows[b], acc_sh.at[ebuf.at[0, 1]], ssem[b]).wait()

        def deg_update(slot):
            if not with_deg:
                return
            for k in range(CHUNK // 16):
                v = ebuf[slot, 1, pl.ds(16 * k, 16)]
                plsc.addupdate_scatter(degl, [v >> 4, v & 15], ones16)

        def emit_body(j, ph, do_ws, do_i, do_g):
            b = ph & 1
            nb = 1 - b
            g_wait(b)
            if do_g:
                i_wait(isem[nb])
                g_start((ph + 1) % 4, nb)
            if do_i:
                i_start(j + 2, (ph + 2) % 4, isem[b])
            deg_update(ph % 4)
            if do_ws:
                s_wait(nb)
            s_start(ph % 4, b)

        i_start(0, 0, isem0)
        i_start(1, 1, isem1)
        i_wait(isem0)
        g_start(0, 0)
        emit_body(0, 0, False, True, True)
        emit_body(1, 1, True, True, True)
        emit_body(2, 2, True, True, True)

        def body(g, carry):
            jb = 3 + 4 * g
            for k in range(4):
                emit_body(jb + k, 3 + k, True, True, True)
            return carry

        lax.fori_loop(0, (RPT - 3 - 4) // 4, body, 0)

        emit_body(RPT - 4, 3, True, True, True)
        emit_body(RPT - 3, 0, True, True, True)
        emit_body(RPT - 2, 1, True, False, True)
        emit_body(RPT - 1, 2, True, False, False)
        s_wait((RPT - 1) & 1)

        if with_deg:
            pltpu.sync_copy(degl, deg_sh.at[iota_v], add=True)
        plsc.subcore_barrier()

        pltpu.sync_copy(acc_sh.at[pl.ds(s * ROWS_PER_TILE, ROWS_PER_TILE)],
                        out_hbm.at[pl.ds(c * NPP + s * ROWS_PER_TILE, ROWS_PER_TILE)])
        if with_deg:
            pltpu.sync_copy(deg_sh.at[pl.ds(s * (DEGR // 16), DEGR // 16)],
                            deg_hbm.at[pl.ds(c * DEGR + s * (DEGR // 16), DEGR // 16)])

    return sc_agg


_sc_agg_deg = _make_sc_agg(True)
_sc_agg_plain = _make_sc_agg(False)

_BLK = 1000


def _tc_layer1(x, p, d0, d1, w_self, w_neigh, b):
    def body(x_ref, p0_ref, p1_ref, d0_ref, d1_ref, ws_ref, wn_ref, b_ref,
             h_ref, inv_ref):
        agg = p0_ref[0] + p1_ref[0]
        deg = d0_ref[...] + d1_ref[...]
        inv = 1.0 / jnp.maximum(deg, 1.0)
        hn = agg * inv
        h = (jnp.dot(x_ref[...], ws_ref[...], preferred_element_type=jnp.float32)
             + jnp.dot(hn, wn_ref[...], preferred_element_type=jnp.float32)
             + b_ref[...])
        h_ref[...] = jnp.maximum(h, 0.0)
        inv_ref[...] = jnp.broadcast_to(inv, (_BLK, 8))

    return pl.pallas_call(
        body,
        grid=(N // _BLK,),
        in_specs=[
            pl.BlockSpec((_BLK, D), lambda i: (i, 0)),
            pl.BlockSpec((1, _BLK, D), lambda i: (0, i, 0)),
            pl.BlockSpec((1, _BLK, D), lambda i: (1, i, 0)),
            pl.BlockSpec((_BLK, 1), lambda i: (i, 0)),
            pl.BlockSpec((_BLK, 1), lambda i: (i, 0)),
            pl.BlockSpec((D, D), lambda i: (0, 0)),
            pl.BlockSpec((D, D), lambda i: (0, 0)),
            pl.BlockSpec((1, D), lambda i: (0, 0)),
        ],
        out_specs=[
            pl.BlockSpec((_BLK, D), lambda i: (i, 0)),
            pl.BlockSpec((_BLK, 8), lambda i: (i, 0)),
        ],
        out_shape=[
            jax.ShapeDtypeStruct((N, D), jnp.float32),
            jax.ShapeDtypeStruct((N, 8), jnp.float32),
        ],
    )(x, p, p, d0, d1, w_self, w_neigh, b)


def _tc_layer2(h, q, inv, w_self, w_neigh, b):
    def body(h_ref, q0_ref, q1_ref, inv_ref, ws_ref, wn_ref, b_ref, o_ref):
        agg = q0_ref[0] + q1_ref[0]
        hn = agg * inv_ref[:, 0:1]
        o_ref[...] = (jnp.dot(h_ref[...], ws_ref[...], preferred_element_type=jnp.float32)
                      + jnp.dot(hn, wn_ref[...], preferred_element_type=jnp.float32)
                      + b_ref[...])

    return pl.pallas_call(
        body,
        grid=(N // _BLK,),
        in_specs=[
            pl.BlockSpec((_BLK, D), lambda i: (i, 0)),
            pl.BlockSpec((1, _BLK, D), lambda i: (0, i, 0)),
            pl.BlockSpec((1, _BLK, D), lambda i: (1, i, 0)),
            pl.BlockSpec((_BLK, 8), lambda i: (i, 0)),
            pl.BlockSpec((D, D), lambda i: (0, 0)),
            pl.BlockSpec((D, D), lambda i: (0, 0)),
            pl.BlockSpec((1, D), lambda i: (0, 0)),
        ],
        out_specs=pl.BlockSpec((_BLK, D), lambda i: (i, 0)),
        out_shape=jax.ShapeDtypeStruct((N, D), jnp.float32),
    )(h, q, q, inv, w_self, w_neigh, b)


def kernel(x, edge_index, W_self1, W_neigh1, b1, W_self2, W_neigh2, b2):
    src = edge_index[0].astype(jnp.int32)
    dst = edge_index[1].astype(jnp.int32)
    ord2 = jnp.argsort(src)
    src = src[ord2]
    dst = dst[ord2]

    pad = E_PAD - E
    pad_dst = N + (jnp.arange(pad, dtype=jnp.int32) % (NPP - N))
    srcm = jnp.concatenate([src, jnp.zeros((pad,), jnp.int32)]).reshape(NCHP, CHUNK)
    dstm = jnp.concatenate([dst, pad_dst]).reshape(NCHP, CHUNK)
    edges = jnp.stack([srcm, dstm], axis=1)
    order = (jnp.arange(NCHP) // RPT) + NTILES * (jnp.arange(NCHP) % RPT)
    edges = edges[order]

    zeros = jnp.zeros((ROWS_PER_TILE, D), jnp.float32)
    zdeg = jnp.zeros((DEGR, 16), jnp.float32)
    iota = jnp.arange(DEGR, dtype=jnp.int32)

    p, od = _sc_agg_deg(x, edges, zeros, zdeg, iota)
    p = p.reshape(2, NPP, D)
    od = od.reshape(2, NPP, 1)
    h, inv = _tc_layer1(x, p, od[0], od[1], W_self1, W_neigh1, b1.reshape(1, D))

    q = _sc_agg_plain(h, edges, zeros, zdeg, iota)[0].reshape(2, NPP, D)
    out = _tc_layer2(h, q, inv, W_self2, W_neigh2, b2.reshape(1, D))
    return out


# speedup vs baseline: 2.1754x; 2.1754x over previous
import functools

import jax
import jax.numpy as jnp
from jax import lax
from jax.experimental import pallas as pl
from jax.experimental.pallas import tpu as pltpu
from jax.experimental.pallas import tpu_sc as plsc

N = 10000
NPP = 10240
E = 320000
D = 128
CHUNK = 128
NTILES = 32
RPT = 79
NCHP = NTILES * RPT
E_PAD = NCHP * CHUNK
ROWS_PER_TILE = NPP // 16
DEGR = NPP // 16


def _make_sc_agg(with_deg):
    mesh = plsc.VectorSubcoreMesh(core_axis_name="c", subcore_axis_name="s",
                                  num_cores=2, num_subcores=16)

    out_type = [jax.ShapeDtypeStruct((2 * NPP, D), jnp.float32)]
    scratch = [
        pltpu.VMEM((4, CHUNK), jnp.int32),
        pltpu.VMEM((4, CHUNK), jnp.int32),
        pltpu.VMEM((CHUNK, D), jnp.float32),
        pltpu.VMEM((CHUNK, D), jnp.float32),
        pltpu.VMEM_SHARED((NPP, D), jnp.float32),
        pltpu.SemaphoreType.DMA,
        pltpu.SemaphoreType.DMA,
        pltpu.SemaphoreType.DMA,
        pltpu.SemaphoreType.DMA,
        pltpu.SemaphoreType.DMA,
        pltpu.SemaphoreType.DMA,
    ]
    if with_deg:
        out_type.append(jax.ShapeDtypeStruct((2 * DEGR, 16), jnp.float32))
        scratch.append(pltpu.VMEM((DEGR, 16), jnp.float32))
        scratch.append(pltpu.VMEM((DEGR,), jnp.int32))
        scratch.append(pltpu.VMEM_SHARED((DEGR, 16), jnp.float32))

    @functools.partial(pl.kernel, out_type=out_type, mesh=mesh,
                       scratch_types=scratch,
                       compiler_params=pltpu.CompilerParams(
                           use_tc_tiling_on_sc=False,
                           needs_layout_passes=False))
    def sc_agg(x_hbm, srcm_hbm, dstm_hbm, zeros_hbm, zdeg_hbm, iota_hbm, *rest):
        if with_deg:
            (out_hbm, deg_hbm, sbuf, dbuf, rows0, rows1, acc_sh,
             gsem0, gsem1, ssem0, ssem1, isem0, isem1,
             degl, iota_v, deg_sh) = rest
        else:
            (out_hbm, sbuf, dbuf, rows0, rows1, acc_sh,
             gsem0, gsem1, ssem0, ssem1, isem0, isem1) = rest
        c = lax.axis_index("c")
        s = lax.axis_index("s")
        wid = c * 16 + s

        pltpu.sync_copy(zeros_hbm, acc_sh.at[pl.ds(s * ROWS_PER_TILE, ROWS_PER_TILE)])
        if with_deg:
            pltpu.sync_copy(zdeg_hbm, degl)
            pltpu.sync_copy(iota_hbm, iota_v)
            pltpu.sync_copy(zdeg_hbm.at[pl.ds(s * (DEGR // 16), DEGR // 16)],
                            deg_sh.at[pl.ds(s * (DEGR // 16), DEGR // 16)])
        plsc.subcore_barrier()

        rows = (rows0, rows1)
        gsem = (gsem0, gsem1)
        ssem = (ssem0, ssem1)
        isem = (isem0, isem1)
        ones16 = jnp.full((16,), 1.0, jnp.float32)

        def i_start(j, slot, sem):
            ci = wid + j * NTILES
            pltpu.async_copy(srcm_hbm.at[ci], sbuf.at[slot], sem)
            pltpu.async_copy(dstm_hbm.at[ci], dbuf.at[slot], sem)

        def i_wait(sem):
            pltpu.make_async_copy(srcm_hbm.at[0], sbuf.at[0], sem).wait()
            pltpu.make_async_copy(dstm_hbm.at[0], dbuf.at[0], sem).wait()

        def g_start(slot, b):
            pltpu.async_copy(x_hbm.at[sbuf.at[slot]], rows[b], gsem[b])

        def g_wait(b):
            pltpu.make_async_copy(x_hbm.at[sbuf.at[0]], rows[b], gsem[b]).wait()

        def s_start(slot, b):
            pltpu.async_copy(rows[b], acc_sh.at[dbuf.at[slot]], ssem[b], add=True)

        def s_wait(b):
            pltpu.make_async_copy(rows[b], acc_sh.at[dbuf.at[0]], ssem[b]).wait()

        def deg_update(slot):
            if not with_deg:
                return
            for k in range(CHUNK // 16):
                v = dbuf[slot, pl.ds(16 * k, 16)]
                plsc.addupdate_scatter(degl, [v >> 4, v & 15], ones16)

        def emit_body(j, ph, do_ws, do_i, do_g):
            b = ph & 1
            nb = 1 - b
            g_wait(b)
            if do_g:
                i_wait(isem[nb])
                g_start((ph + 1) % 4, nb)
            if do_i:
                i_start(j + 2, (ph + 2) % 4, isem[b])
            deg_update(ph % 4)
            if do_ws:
                s_wait(nb)
            s_start(ph % 4, b)

        i_start(0, 0, isem0)
        i_start(1, 1, isem1)
        i_wait(isem0)
        g_start(0, 0)
        emit_body(0, 0, False, True, True)
        emit_body(1, 1, True, True, True)
        emit_body(2, 2, True, True, True)

        def body(g, carry):
            jb = 3 + 4 * g
            for k in range(4):
                emit_body(jb + k, 3 + k, True, True, True)
            return carry

        lax.fori_loop(0, (RPT - 3 - 4) // 4, body, 0)

        emit_body(RPT - 4, 3, True, True, True)
        emit_body(RPT - 3, 0, True, True, True)
        emit_body(RPT - 2, 1, True, False, True)
        emit_body(RPT - 1, 2, True, False, False)
        s_wait((RPT - 1) & 1)

        if with_deg:
            pltpu.sync_copy(degl, deg_sh.at[iota_v], add=True)
        plsc.subcore_barrier()

        pltpu.sync_copy(acc_sh.at[pl.ds(s * ROWS_PER_TILE, ROWS_PER_TILE)],
                        out_hbm.at[pl.ds(c * NPP + s * ROWS_PER_TILE, ROWS_PER_TILE)])
        if with_deg:
            pltpu.sync_copy(deg_sh.at[pl.ds(s * (DEGR // 16), DEGR // 16)],
                            deg_hbm.at[pl.ds(c * DEGR + s * (DEGR // 16), DEGR // 16)])

    return sc_agg


_sc_agg_deg = _make_sc_agg(True)
_sc_agg_plain = _make_sc_agg(False)

_BLK = 1000


def _tc_layer1(x, p, d0, d1, w_self, w_neigh, b):
    def body(x_ref, p0_ref, p1_ref, d0_ref, d1_ref, ws_ref, wn_ref, b_ref,
             h_ref, inv_ref):
        agg = p0_ref[0] + p1_ref[0]
        deg = d0_ref[...] + d1_ref[...]
        inv = 1.0 / jnp.maximum(deg, 1.0)
        hn = agg * inv
        h = (jnp.dot(x_ref[...], ws_ref[...], preferred_element_type=jnp.float32)
             + jnp.dot(hn, wn_ref[...], preferred_element_type=jnp.float32)
             + b_ref[...])
        h_ref[...] = jnp.maximum(h, 0.0)
        inv_ref[...] = jnp.broadcast_to(inv, (_BLK, 8))

    return pl.pallas_call(
        body,
        grid=(N // _BLK,),
        in_specs=[
            pl.BlockSpec((_BLK, D), lambda i: (i, 0)),
            pl.BlockSpec((1, _BLK, D), lambda i: (0, i, 0)),
            pl.BlockSpec((1, _BLK, D), lambda i: (1, i, 0)),
            pl.BlockSpec((_BLK, 1), lambda i: (i, 0)),
            pl.BlockSpec((_BLK, 1), lambda i: (i, 0)),
            pl.BlockSpec((D, D), lambda i: (0, 0)),
            pl.BlockSpec((D, D), lambda i: (0, 0)),
            pl.BlockSpec((1, D), lambda i: (0, 0)),
        ],
        out_specs=[
            pl.BlockSpec((_BLK, D), lambda i: (i, 0)),
            pl.BlockSpec((_BLK, 8), lambda i: (i, 0)),
        ],
        out_shape=[
            jax.ShapeDtypeStruct((N, D), jnp.float32),
            jax.ShapeDtypeStruct((N, 8), jnp.float32),
        ],
    )(x, p, p, d0, d1, w_self, w_neigh, b)


def _tc_layer2(h, q, inv, w_self, w_neigh, b):
    def body(h_ref, q0_ref, q1_ref, inv_ref, ws_ref, wn_ref, b_ref, o_ref):
        agg = q0_ref[0] + q1_ref[0]
        hn = agg * inv_ref[:, 0:1]
        o_ref[...] = (jnp.dot(h_ref[...], ws_ref[...], preferred_element_type=jnp.float32)
                      + jnp.dot(hn, wn_ref[...], preferred_element_type=jnp.float32)
                      + b_ref[...])

    return pl.pallas_call(
        body,
        grid=(N // _BLK,),
        in_specs=[
            pl.BlockSpec((_BLK, D), lambda i: (i, 0)),
            pl.BlockSpec((1, _BLK, D), lambda i: (0, i, 0)),
            pl.BlockSpec((1, _BLK, D), lambda i: (1, i, 0)),
            pl.BlockSpec((_BLK, 8), lambda i: (i, 0)),
            pl.BlockSpec((D, D), lambda i: (0, 0)),
            pl.BlockSpec((D, D), lambda i: (0, 0)),
            pl.BlockSpec((1, D), lambda i: (0, 0)),
        ],
        out_specs=pl.BlockSpec((_BLK, D), lambda i: (i, 0)),
        out_shape=jax.ShapeDtypeStruct((N, D), jnp.float32),
    )(h, q, q, inv, w_self, w_neigh, b)


def kernel(x, edge_index, W_self1, W_neigh1, b1, W_self2, W_neigh2, b2):
    src = edge_index[0].astype(jnp.int32)
    dst = edge_index[1].astype(jnp.int32)

    pad = E_PAD - E
    pad_dst = N + (jnp.arange(pad, dtype=jnp.int32) % (NPP - N))
    srcm = jnp.concatenate([src, jnp.zeros((pad,), jnp.int32)]).reshape(NCHP, CHUNK)
    dstm = jnp.concatenate([dst, pad_dst]).reshape(NCHP, CHUNK)

    zeros = jnp.zeros((ROWS_PER_TILE, D), jnp.float32)
    zdeg = jnp.zeros((DEGR, 16), jnp.float32)
    iota = jnp.arange(DEGR, dtype=jnp.int32)

    p, od = _sc_agg_deg(x, srcm, dstm, zeros, zdeg, iota)
    p = p.reshape(2, NPP, D)
    od = od.reshape(2, NPP, 1)
    h, inv = _tc_layer1(x, p, od[0], od[1], W_self1, W_neigh1, b1.reshape(1, D))

    q = _sc_agg_plain(h, srcm, dstm, zeros, zdeg, iota)[0].reshape(2, NPP, D)
    out = _tc_layer2(h, q, inv, W_self2, W_neigh2, b2.reshape(1, D))
    return out
